# Initial kernel scaffold; baseline (speedup 1.0000x reference)
import jax
import jax.numpy as jnp
from jax.experimental import pallas as pl


def kernel(features_1, features_2, edge_index_1, edge_index_2, avg_v, eps1, W11, b11, W12, b12, g1, be1, eps2, W21, b21, W22, b22, g2, be2, eps3, W31, b31, W32, b32, g3, be3, Wc, Wc1, bc1, Wc2, bc2, Wm, Wm1, bm1, Wm2, bm2, Wa, Wt, Wtb, bt, Wf1, bf1, Wf2, bf2, Wf3, bf3, Ws, bs):
    raise NotImplementedError("write your pallas kernel here")


# speedup vs baseline: 3.5696x; 3.5696x over previous
import functools

import jax
import jax.numpy as jnp
from jax import lax
from jax.experimental import pallas as pl
from jax.experimental.pallas import tpu as pltpu
from jax.experimental.pallas import tpu_sc as plsc

N = 1024
E = 16384
F1, F2, F3 = 128, 64, 32
NTILES = 16
EDGES_PER_TILE = E // NTILES
CHUNK = 128
NCHUNK = EDGES_PER_TILE // CHUNK
ROWS_PER_TILE = N // NTILES

BN = 128
GRP = 8
NG = BN // GRP

_HI = jax.lax.Precision.HIGHEST


def _f32dot(a, b, dims=None):
    if dims is None:
        return jnp.dot(a, b, precision=_HI, preferred_element_type=jnp.float32)
    return lax.dot_general(a, b, (dims, ((), ())), precision=_HI,
                           preferred_element_type=jnp.float32)


def _round_bf16(x):
    return x.astype(jnp.bfloat16).astype(jnp.float32)


def _sc_agg(x_both, ei_both, zeros_both):
    mesh = plsc.VectorSubcoreMesh(core_axis_name="c", subcore_axis_name="s")

    @functools.partial(
        pl.kernel,
        out_type=jax.ShapeDtypeStruct((2, N, F1), jnp.float32),
        mesh=mesh,
        scratch_types=[
            pltpu.VMEM((NCHUNK, CHUNK), jnp.int32),
            pltpu.VMEM((NCHUNK, CHUNK), jnp.int32),
            pltpu.VMEM((CHUNK, F1), jnp.float32),
            pltpu.VMEM_SHARED((N, F1), jnp.float32),
            pltpu.SemaphoreType.DMA,
        ],
    )
    def agg(x_hbm, ei_hbm, z_hbm, out_hbm, src_v, dst_v, rows_v, acc_sh, sem):
        c = lax.axis_index("c")
        s = lax.axis_index("s")
        rbase = s * ROWS_PER_TILE
        pltpu.sync_copy(z_hbm.at[c, pl.ds(rbase, ROWS_PER_TILE)],
                        acc_sh.at[pl.ds(rbase, ROWS_PER_TILE)])
        pltpu.sync_copy(ei_hbm.at[c, 0, s], src_v)
        pltpu.sync_copy(ei_hbm.at[c, 1, s], dst_v)
        plsc.subcore_barrier()
        for ch in range(NCHUNK):
            pltpu.async_copy(x_hbm.at[c].at[src_v.at[ch]], rows_v, sem).wait()
            pltpu.sync_copy(rows_v, acc_sh.at[dst_v.at[ch]], add=True)
        plsc.subcore_barrier()
        pltpu.sync_copy(acc_sh.at[pl.ds(rbase, ROWS_PER_TILE)],
                        out_hbm.at[c, pl.ds(rbase, ROWS_PER_TILE)])

    return agg(x_both, ei_both, zeros_both)


_USE_SC = True


def _aggregate(x_both, ei_r, zeros):
    if _USE_SC:
        return _sc_agg(x_both, ei_r, zeros)
    out = []
    for g in range(2):
        src = ei_r[g, 0].reshape(-1)
        dst = ei_r[g, 1].reshape(-1)
        out.append(jnp.zeros((N, F1), jnp.float32)
                   .at[dst].add(x_both[g][src]))
    return jnp.stack(out)


def _dense_body(fin, fout, x_ref, s_ref, eps_ref, ba_ref, wa_ref, wbb_ref,
                bb_ref, g_ref, be_ref, o_ref):
    eps = eps_ref[0]
    for g in range(2):
        xc = (1.0 + eps) * x_ref[g, :, :fin] + s_ref[g, :, :fin]
        z = jnp.maximum(_f32dot(xc, wa_ref[...]) + ba_ref[...], 0.0)
        h = _f32dot(z, wbb_ref[...]) + bb_ref[...]
        mu = jnp.mean(h, axis=0, keepdims=True)
        var = jnp.mean((h - mu) ** 2, axis=0, keepdims=True)
        hn = (h - mu) / jnp.sqrt(var + 1e-5) * g_ref[...] + be_ref[...]
        x = jnp.maximum(hn, 0.0)
        if fout == F1:
            o_ref[g] = x
        else:
            o_ref[g] = jnp.concatenate(
                [x, jnp.zeros((N, F1 - fout), jnp.float32)], axis=1)


def _dense_layer(x, s, eps, ba, Wa, Wb, bb, gam, bet, fin, fout):
    return pl.pallas_call(
        functools.partial(_dense_body, fin, fout),
        out_shape=jax.ShapeDtypeStruct((2, N, F1), jnp.float32),
        in_specs=[
            pl.BlockSpec(), pl.BlockSpec(),
            pl.BlockSpec(memory_space=pltpu.SMEM),
            pl.BlockSpec(), pl.BlockSpec(), pl.BlockSpec(),
            pl.BlockSpec(), pl.BlockSpec(), pl.BlockSpec(),
        ],
    )(x, s, eps, ba, Wa, Wb, bb, gam, bet)


def _final_body(x_ref, s_ref, eps_ref, ba_ref, wa_ref, wbb_ref, bb_ref,
                g_ref, be_ref, wall_ref, watt_ref,
                t_ref, a2b_ref, e_ref):
    eps = eps_ref[0]
    embs = []
    for g in range(2):
        xc = (1.0 + eps) * x_ref[g, :, :F2] + s_ref[g, :, :F2]
        z = jnp.maximum(_f32dot(xc, wa_ref[...]) + ba_ref[...], 0.0)
        h = _f32dot(z, wbb_ref[...]) + bb_ref[...]
        mu = jnp.mean(h, axis=0, keepdims=True)
        var = jnp.mean((h - mu) ** 2, axis=0, keepdims=True)
        a = (h - mu) / jnp.sqrt(var + 1e-5) * g_ref[...] + be_ref[...]
        embs.append(a)
        ga = _f32dot(a, watt_ref[...])
        gc = jnp.tanh(jnp.mean(ga, axis=0, keepdims=True))
        sig = jax.nn.sigmoid(_f32dot(a, gc, dims=((1,), (1,))))
        e_ref[g] = _f32dot(sig, a, dims=((0,), (0,)))[0]
    t = jnp.dot(embs[0].astype(jnp.bfloat16), wall_ref[...],
                preferred_element_type=jnp.float32)
    t_ref[...] = t.astype(jnp.bfloat16)
    a2b_ref[...] = embs[1].astype(jnp.bfloat16)


def _final_layer(x, s, eps, ba, Wa, Wb, bb, gam, bet, Wall, Watt):
    return pl.pallas_call(
        _final_body,
        out_shape=[
            jax.ShapeDtypeStruct((N, 1024), jnp.bfloat16),
            jax.ShapeDtypeStruct((N, F3), jnp.bfloat16),
            jax.ShapeDtypeStruct((2, F3), jnp.float32),
        ],
        in_specs=[
            pl.BlockSpec(), pl.BlockSpec(),
            pl.BlockSpec(memory_space=pltpu.SMEM),
            pl.BlockSpec(), pl.BlockSpec(), pl.BlockSpec(),
            pl.BlockSpec(), pl.BlockSpec(), pl.BlockSpec(),
            pl.BlockSpec(), pl.BlockSpec(),
        ],
    )(x, s, eps, ba, Wa, Wb, bb, gam, bet, Wall, Watt)


def _pair_body(t_ref, a2b_ref, s3_ref, s4_ref, c1_ref, c2_ref,
               mapm_ref, tot_ref, t2_scr, cost_scr):
    i = pl.program_id(0)
    a2b = a2b_ref[...]
    for h in range(2):
        for k in range(16):
            tk = t_ref[:, (h * 16 + k) * 32:(h * 16 + k + 1) * 32]
            u = lax.dot_general(tk, a2b, (((1,), (1,)), ((), ())),
                                preferred_element_type=jnp.float32)
            t2_scr[k] = u.astype(jnp.bfloat16)
        for gi in range(NG):
            blk = t2_scr[:, gi * GRP:(gi + 1) * GRP, :]
            blk = jnp.reshape(blk, (16 * GRP, 1024))
            hpre = jnp.dot(s3_ref[h], blk,
                           preferred_element_type=jnp.float32) + c1_ref[h]
            hb = jnp.maximum(hpre, 0.0).astype(jnp.bfloat16)
            og = jnp.dot(s4_ref[h], hb,
                         preferred_element_type=jnp.float32) + c2_ref[0, h]
            if h == 0:
                cost_scr[gi * GRP:(gi + 1) * GRP, :] = og
            else:
                mapm_ref[gi * GRP:(gi + 1) * GRP, :] = og
    m = mapm_ref[...]
    rmax = jnp.max(m, axis=1, keepdims=True)
    p = jnp.exp(m - rmax)
    denom = jnp.sum(p, axis=1, keepdims=True)
    num = jnp.sum(p * cost_scr[...], axis=1, keepdims=True)
    part = jnp.sum(num / denom, axis=0, keepdims=True)

    @pl.when(i == 0)
    def _init():
        tot_ref[...] = part

    @pl.when(i > 0)
    def _acc():
        tot_ref[...] += part


def _pairwise(t, a2b, S3, S4, c1col, c2s):
    return pl.pallas_call(
        _pair_body,
        grid=(N // BN,),
        out_shape=[
            jax.ShapeDtypeStruct((N, N), jnp.float32),
            jax.ShapeDtypeStruct((1, 1), jnp.float32),
        ],
        in_specs=[
            pl.BlockSpec((BN, 1024), lambda i: (i, 0)),
            pl.BlockSpec((N, F3), lambda i: (0, 0)),
            pl.BlockSpec((2, 32 * GRP, 16 * GRP), lambda i: (0, 0, 0)),
            pl.BlockSpec((2, GRP, 32 * GRP), lambda i: (0, 0, 0)),
            pl.BlockSpec((2, 32 * GRP, 1), lambda i: (0, 0, 0)),
            pl.BlockSpec(memory_space=pltpu.SMEM),
        ],
        out_specs=[
            pl.BlockSpec((BN, N), lambda i: (i, 0)),
            pl.BlockSpec((1, 1), lambda i: (0, 0)),
        ],
        scratch_shapes=[
            pltpu.VMEM((16, BN, 1024), jnp.bfloat16),
            pltpu.VMEM((BN, N), jnp.float32),
        ],
    )(t, a2b, S3, S4, c1col, c2s)


def _head_body(e_ref, wt_ref, wtba_ref, wtbb_ref, bt_ref, wf1_ref, bf1_ref,
               wf2_ref, bf2_ref, wf3_ref, bf3_ref, ws_ref, bs_ref, tot_ref,
               av_ref, score_ref, ged_ref):
    e1 = e_ref[0:1, :]
    e2 = e_ref[1:2, :]
    parts = []
    for t in range(16):
        w = _f32dot(e1, wt_ref[t])
        parts.append(jnp.sum(w * e2, axis=1, keepdims=True))
    scoring = jnp.concatenate(parts, axis=1)
    block = _f32dot(e1, wtba_ref[...]) + _f32dot(e2, wtbb_ref[...])
    s = jnp.maximum(scoring + block + bt_ref[...], 0.0)
    s = jnp.maximum(_f32dot(s, wf1_ref[...]) + bf1_ref[...], 0.0)
    s = jnp.maximum(_f32dot(s, wf2_ref[...]) + bf2_ref[...], 0.0)
    s = jnp.maximum(_f32dot(s, wf3_ref[...]) + bf3_ref[...], 0.0)
    bias = _f32dot(s, ws_ref[...]) + bs_ref[...]
    score = jax.nn.sigmoid(tot_ref[...] + bias)
    score_ref[...] = score
    ged_ref[...] = -jnp.log(score) * av_ref[...]


def _head(e, Wt_t, Wtb_a, Wtb_b, bt, Wf1, bf1, Wf2, bf2, Wf3, bf3, Ws, bs,
          tot, av):
    return pl.pallas_call(
        _head_body,
        out_shape=[
            jax.ShapeDtypeStruct((1, 1), jnp.float32),
            jax.ShapeDtypeStruct((1, 1), jnp.float32),
        ],
    )(e, Wt_t, Wtb_a, Wtb_b, bt, Wf1, bf1, Wf2, bf2, Wf3, bf3, Ws, bs, tot, av)


def kernel(features_1, features_2, edge_index_1, edge_index_2, avg_v,
           eps1, W11, b11, W12, b12, g1, be1,
           eps2, W21, b21, W22, b22, g2, be2,
           eps3, W31, b31, W32, b32, g3, be3,
           Wc, Wc1, bc1, Wc2, bc2,
           Wm, Wm1, bm1, Wm2, bm2,
           Wa, Wt, Wtb, bt,
           Wf1, bf1, Wf2, bf2, Wf3, bf3, Ws, bs):
    r1 = lambda v: v.reshape(1, -1)

    f_both = jnp.stack([features_1, features_2])
    ei_both = jnp.stack([edge_index_1, edge_index_2])
    ei_r = ei_both.reshape(2, 2, NTILES, NCHUNK, CHUNK)
    agg_zeros = jnp.zeros((2, N, F1), jnp.float32)

    Wall = jnp.concatenate(
        [Wc.transpose(1, 0, 2).reshape(F3, 512),
         Wm.transpose(1, 0, 2).reshape(F3, 512)],
        axis=1).astype(jnp.bfloat16)
    eye = jnp.eye(GRP, dtype=jnp.float32)
    S3 = jnp.stack([
        jnp.einsum('kj,np->njkp', M1h, eye).reshape(32 * GRP, 16 * GRP)
        for M1h in (Wc1, Wm1)]).astype(jnp.bfloat16)
    S4 = jnp.stack([
        jnp.einsum('np,j->npj', eye, M2h[:, 0]).reshape(GRP, 32 * GRP)
        for M2h in (Wc2, Wm2)]).astype(jnp.bfloat16)
    c1col = jnp.stack([jnp.tile(bc1, GRP)[:, None],
                       jnp.tile(bm1, GRP)[:, None]])
    c2s = jnp.stack([bc2, bm2], axis=1)

    eps1r = eps1.reshape(1)
    eps2r = eps2.reshape(1)
    eps3r = eps3.reshape(1)

    s1 = _aggregate(f_both, ei_r, agg_zeros)
    x2 = _dense_layer(f_both, s1, eps1r, r1(b11), W11, W12, r1(b12),
                      r1(g1), r1(be1), F1, F1)
    s2 = _aggregate(x2, ei_r, agg_zeros)
    x3 = _dense_layer(x2, s2, eps2r, r1(b21), W21, W22, r1(b22),
                      r1(g2), r1(be2), F1, F2)
    s3 = _aggregate(x3, ei_r, agg_zeros)
    t, a2b, e_both = _final_layer(x3, s3, eps3r, r1(b31), W31, W32, r1(b32),
                                  r1(g3), r1(be3), Wall, Wa)

    mapm, tot = _pairwise(t, a2b, S3, S4, c1col, c2s)

    Wt_t = jnp.transpose(Wt, (2, 0, 1))
    score, ged = _head(e_both, Wt_t, Wtb[:, :F3].T, Wtb[:, F3:].T, r1(bt),
                       Wf1, r1(bf1), Wf2, r1(bf2), Wf3, r1(bf3), Ws, r1(bs),
                       tot, avg_v.reshape(1, 1))
    return (score.reshape(1), ged.reshape(1), mapm)
